# baseline (speedup 1.0000x reference)
import functools

import jax
import jax.numpy as jnp
from jax import lax
from jax.experimental import pallas as pl
from jax.experimental.pallas import tpu as pltpu
from jax.experimental.pallas import tpu_sc as plsc

NUM_CORES = 2
NUM_SUBCORES = 16
NUM_WORKERS = NUM_CORES * NUM_SUBCORES
LANES = 16

BATCH = 16384
DIM = 32
ROWS_PER_WORKER = BATCH // NUM_WORKERS
IDX_ROWS = 4
IDX_COLS = ROWS_PER_WORKER // IDX_ROWS
CHUNKS = ROWS_PER_WORKER // LANES


def _gmf_body(user_table, item_table, w_hbm, b_hbm, uidx_hbm, iidx_hbm,
              out_hbm, uidx_v, iidx_v, rows_u, rows_i, w_v, b_v, out_v, sem):
    wid = lax.axis_index("s") * NUM_CORES + lax.axis_index("c")
    base = wid * ROWS_PER_WORKER

    pltpu.sync_copy(uidx_hbm.at[pl.ds(wid * IDX_ROWS, IDX_ROWS)], uidx_v)
    pltpu.sync_copy(iidx_hbm.at[pl.ds(wid * IDX_ROWS, IDX_ROWS)], iidx_v)
    pltpu.sync_copy(w_hbm, w_v)
    pltpu.sync_copy(b_hbm, b_v)

    copies = []
    for j in range(IDX_ROWS):
        dst = pl.ds(j * IDX_COLS, IDX_COLS)
        copies.append(pltpu.async_copy(
            user_table.at[uidx_v.at[j]], rows_u.at[dst], sem))
        copies.append(pltpu.async_copy(
            item_table.at[iidx_v.at[j]], rows_i.at[dst], sem))
    for c in copies:
        c.wait()

    iota = lax.iota(jnp.int32, LANES)
    bias = b_v[...]
    w_lo = w_v[pl.ds(0, LANES)]
    w_hi = w_v[pl.ds(LANES, LANES)]

    def chunk(c, carry):
        rid = c * LANES + iota
        acc = bias
        for d in range(DIM):
            col = jnp.full((LANES,), d, dtype=jnp.int32)
            ug = plsc.load_gather(rows_u, [rid, col])
            vg = plsc.load_gather(rows_i, [rid, col])
            w_s = (w_lo if d < LANES else w_hi)[d % LANES]
            acc = acc + ug * vg * w_s
        out_v[pl.ds(c * LANES, LANES)] = 1.0 / (1.0 + jnp.exp(-acc))
        return carry

    lax.fori_loop(0, CHUNKS, chunk, 0)

    pltpu.sync_copy(out_v, out_hbm.at[pl.ds(base, ROWS_PER_WORKER)])


@functools.partial(jax.jit, static_argnames=())
def _gmf(user_table, item_table, w_flat, b_vec, uidx2d, iidx2d):
    mesh = plsc.VectorSubcoreMesh(core_axis_name="c", subcore_axis_name="s")
    run = pl.kernel(
        _gmf_body,
        out_type=jax.ShapeDtypeStruct((BATCH,), jnp.float32),
        mesh=mesh,
        scratch_types=[
            pltpu.VMEM((IDX_ROWS, IDX_COLS), jnp.int32),
            pltpu.VMEM((IDX_ROWS, IDX_COLS), jnp.int32),
            pltpu.VMEM((ROWS_PER_WORKER, DIM), jnp.float32),
            pltpu.VMEM((ROWS_PER_WORKER, DIM), jnp.float32),
            pltpu.VMEM((DIM,), jnp.float32),
            pltpu.VMEM((LANES,), jnp.float32),
            pltpu.VMEM((ROWS_PER_WORKER,), jnp.float32),
            pltpu.SemaphoreType.DMA,
        ],
        compiler_params=pltpu.CompilerParams(
            needs_layout_passes=False, use_tc_tiling_on_sc=False),
    )
    return run(user_table, item_table, w_flat, b_vec, uidx2d, iidx2d)


def kernel(user_table, item_table, fc_w, fc_b, user_indices, item_indices):
    w_flat = fc_w.reshape(DIM).astype(jnp.float32)
    b_vec = jnp.broadcast_to(fc_b.astype(jnp.float32), (LANES,))
    uidx2d = user_indices.astype(jnp.int32).reshape(NUM_WORKERS * IDX_ROWS,
                                                    IDX_COLS)
    iidx2d = item_indices.astype(jnp.int32).reshape(NUM_WORKERS * IDX_ROWS,
                                                    IDX_COLS)
    out = _gmf(user_table, item_table, w_flat, b_vec, uidx2d, iidx2d)
    return out.reshape(BATCH, 1)
